# speedup vs baseline: 1.5022x; 1.5022x over previous
from math import sqrt

import jax
import jax.numpy as jnp
from jax.experimental import pallas as pl


def _pe_add_kernel(x_ref, pe_ref, out_ref, *, scale):
    out_ref[...] = x_ref[...] * scale + pe_ref[...]


def kernel(x, emb_table):
    batch, seq, d = x.shape
    scale = sqrt(float(d))
    blk_s = 512
    grid = (seq // blk_s, batch)

    return pl.pallas_call(
        lambda x_ref, pe_ref, out_ref: _pe_add_kernel(
            x_ref, pe_ref, out_ref, scale=scale
        ),
        grid=grid,
        in_specs=[
            pl.BlockSpec((1, blk_s, d), lambda i, j: (j, i, 0)),
            pl.BlockSpec((blk_s, d), lambda i, j: (i, 0)),
        ],
        out_specs=pl.BlockSpec((1, blk_s, d), lambda i, j: (j, i, 0)),
        out_shape=jax.ShapeDtypeStruct((batch, seq, d), x.dtype),
    )(x, emb_table[:seq])


# speedup vs baseline: 1.6667x; 1.1095x over previous
from math import sqrt

import jax
import jax.numpy as jnp
from jax.experimental import pallas as pl


def _pe_add_kernel(x_ref, pe_ref, out_ref, *, scale):
    out_ref[...] = x_ref[...] * scale + pe_ref[...]


def kernel(x, emb_table):
    batch, seq, d = x.shape
    scale = sqrt(float(d))
    blk_s = 1024
    grid = (seq // blk_s, batch)

    return pl.pallas_call(
        lambda x_ref, pe_ref, out_ref: _pe_add_kernel(
            x_ref, pe_ref, out_ref, scale=scale
        ),
        grid=grid,
        in_specs=[
            pl.BlockSpec((1, blk_s, d), lambda i, j: (j, i, 0)),
            pl.BlockSpec((blk_s, d), lambda i, j: (i, 0)),
        ],
        out_specs=pl.BlockSpec((1, blk_s, d), lambda i, j: (j, i, 0)),
        out_shape=jax.ShapeDtypeStruct((batch, seq, d), x.dtype),
    )(x, emb_table[:seq])


# speedup vs baseline: 1.7358x; 1.0414x over previous
from math import sqrt

import jax
import jax.numpy as jnp
from jax.experimental import pallas as pl


def _pe_add_kernel(x_ref, pe_ref, out_ref, *, scale):
    out_ref[...] = x_ref[...] * scale + pe_ref[...]


def kernel(x, emb_table):
    batch, seq, d = x.shape
    scale = sqrt(float(d))
    blk_s = 2048
    grid = (seq // blk_s, batch)

    return pl.pallas_call(
        lambda x_ref, pe_ref, out_ref: _pe_add_kernel(
            x_ref, pe_ref, out_ref, scale=scale
        ),
        grid=grid,
        in_specs=[
            pl.BlockSpec((1, blk_s, d), lambda i, j: (j, i, 0)),
            pl.BlockSpec((blk_s, d), lambda i, j: (i, 0)),
        ],
        out_specs=pl.BlockSpec((1, blk_s, d), lambda i, j: (j, i, 0)),
        out_shape=jax.ShapeDtypeStruct((batch, seq, d), x.dtype),
    )(x, emb_table[:seq])


# speedup vs baseline: 1.7380x; 1.0013x over previous
from math import sqrt

import jax
import jax.numpy as jnp
from jax.experimental import pallas as pl
from jax.experimental.pallas import tpu as pltpu


def _pe_add_kernel(x_ref, pe_ref, out_ref, *, scale):
    out_ref[...] = x_ref[...] * scale + pe_ref[...]


def kernel(x, emb_table):
    batch, seq, d = x.shape
    scale = sqrt(float(d))
    blk_s = 2048
    grid = (seq // blk_s, batch)

    return pl.pallas_call(
        lambda x_ref, pe_ref, out_ref: _pe_add_kernel(
            x_ref, pe_ref, out_ref, scale=scale
        ),
        grid=grid,
        in_specs=[
            pl.BlockSpec((1, blk_s, d), lambda i, j: (j, i, 0)),
            pl.BlockSpec((blk_s, d), lambda i, j: (i, 0)),
        ],
        out_specs=pl.BlockSpec((1, blk_s, d), lambda i, j: (j, i, 0)),
        out_shape=jax.ShapeDtypeStruct((batch, seq, d), x.dtype),
        compiler_params=pltpu.CompilerParams(
            dimension_semantics=("parallel", "parallel"),
            vmem_limit_bytes=128 * 1024 * 1024,
        ),
    )(x, emb_table[:seq])
